# baseline (speedup 1.0000x reference)
import functools

import jax
import jax.numpy as jnp
from jax import lax
from jax.experimental import pallas as pl
from jax.experimental.pallas import tpu as pltpu
import jax.experimental.pallas.tpu_sc as plsc

MAX_REL = 2048
Q_LEN = 32
K_LEN = 2048
D_MODEL = 1024

NC, NS = 2, 16
NW = NC * NS
LANES = 16

CO = 16
NR = 4
RING_ROWS = NR * CO
NK = K_LEN // CO - 1
NCI = K_LEN // CO
NO = 2
VPR = D_MODEL // LANES


def _sc_body(rel_hbm, out_hbm, eidx_v, in_ring, out_v, erows_v,
             esem, *sems):
    rsems, wsems = sems[:NR], sems[NR:]
    w = lax.axis_index("s") * NC + lax.axis_index("c")
    wp1 = w + 1
    pad = lax.rem(8 - lax.rem(wp1, 8), 8)
    a1 = pl.multiple_of(wp1 + pad, 8)

    eidx_v[pl.ds(0, LANES)] = jnp.full((LANES,), w + CO, jnp.int32) - lax.iota(
        jnp.int32, LANES)
    egather = pltpu.async_copy(rel_hbm.at[eidx_v], erows_v, esem)

    def start_read_slot(ci, s):
        pltpu.async_copy(
            rel_hbm.at[pl.ds(a1 + ci * CO, CO)],
            in_ring.at[pl.ds(s * CO, CO)], rsems[s])

    def wait_read_slot(s):
        pltpu.make_async_copy(
            rel_hbm.at[pl.ds(0, CO)], in_ring.at[pl.ds(s * CO, CO)], rsems[s]
        ).wait()

    def start_write(k, o):
        pltpu.async_copy(out_v.at[pl.ds(o * CO, CO)],
                         out_hbm.at[w, pl.ds(k * CO, CO)], wsems[o])

    def wait_write(o):
        pltpu.make_async_copy(
            out_v.at[pl.ds(o * CO, CO)], out_hbm.at[w, pl.ds(0, CO)], wsems[o]
        ).wait()

    for d in range(NR):
        ci = NCI - 1 - d
        start_read_slot(ci, ci % NR)

    def step(k, carry):
        for s in range(NR):
            @pl.when(lax.rem(NCI - 2 - k, NR) == s)
            def _():
                wait_read_slot(s)

        o_slot = lax.rem(k, NO)
        for o in range(NO):
            @pl.when(jnp.logical_and(o_slot == o, k >= NO))
            def _():
                wait_write(o)

        base = K_LEN - 1 - pad - CO * k
        dst0 = o_slot * CO

        def copy_row(t, c2):
            src = lax.rem(base - t, RING_ROWS)
            dst = dst0 + t
            for v in range(VPR):
                sl = pl.ds(v * LANES, LANES)
                out_v[dst, sl] = in_ring[src, sl]
            return c2

        lax.fori_loop(0, CO, copy_row, 0)

        for o in range(NO):
            @pl.when(o_slot == o)
            def _():
                start_write(k, o)

        nci = NCI - 1 - NR - k
        for s in range(NR):
            @pl.when(jnp.logical_and(lax.rem(NCI - 1 - k, NR) == s, nci >= 0))
            def _():
                start_read_slot(nci, s)

        return carry

    for s in range(NR):
        @pl.when(lax.rem(NCI - 1, NR) == s)
        def _():
            wait_read_slot(s)

    lax.fori_loop(0, NK, step, 0)

    for o in range(NO):
        wait_write(o)
    egather.wait()
    pltpu.sync_copy(erows_v, out_hbm.at[w, pl.ds(K_LEN - CO, CO)])


@functools.partial(jax.jit, static_argnames=())
def _sc_gather(rel_emb):
    mesh = plsc.VectorSubcoreMesh(core_axis_name="c", subcore_axis_name="s")
    run = pl.kernel(
        _sc_body,
        out_type=jax.ShapeDtypeStruct((Q_LEN, K_LEN, D_MODEL), jnp.float32),
        mesh=mesh,
        scratch_types=(
            [pltpu.VMEM((LANES,), jnp.int32),
             pltpu.VMEM((RING_ROWS, D_MODEL), jnp.float32),
             pltpu.VMEM((NO * CO, D_MODEL), jnp.float32),
             pltpu.VMEM((CO, D_MODEL), jnp.float32),
             pltpu.SemaphoreType.DMA]
            + [pltpu.SemaphoreType.DMA] * (NR + NO)
        ),
    )
    return run(rel_emb)


def kernel(q, k, rel_emb):
    del q, k
    return _sc_gather(rel_emb)


# speedup vs baseline: 1.1434x; 1.1434x over previous
import functools

import jax
import jax.numpy as jnp
from jax import lax
from jax.experimental import pallas as pl
from jax.experimental.pallas import tpu as pltpu
import jax.experimental.pallas.tpu_sc as plsc

MAX_REL = 2048
Q_LEN = 32
K_LEN = 2048
D_MODEL = 1024

NC, NS = 2, 16
NW = NC * NS
LANES = 16

CO = 16
NSLOT = 3
NKA = K_LEN // 2 // CO
NKB = NKA + 1
TRASH = Q_LEN * K_LEN


def _sc_body(rel_hbm, out_hbm, bufa, bufb, *sems):
    ga, wa = sems[0:NSLOT], sems[NSLOT:2 * NSLOT]
    rb, sb = sems[2 * NSLOT:3 * NSLOT], sems[3 * NSLOT:4 * NSLOT]
    w = lax.axis_index("s") * NC + lax.axis_index("c")
    row0 = w * K_LEN
    alow = (w + 1) // CO * CO
    iota = lax.iota(jnp.int32, LANES)

    def issue_ga(k, s):
        idx = jnp.full((LANES,), w + MAX_REL, jnp.int32) - k * CO - iota
        pltpu.async_copy(rel_hbm.at[idx], bufa.at[pl.ds(s * CO, CO)], ga[s])

    def issue_wa(k, s):
        off = pl.multiple_of(row0 + k * CO, 8)
        pltpu.async_copy(bufa.at[pl.ds(s * CO, CO)],
                         out_hbm.at[pl.ds(off, CO)], wa[s])

    def issue_rb(k, s):
        off = pl.multiple_of(alow + k * CO, 8)
        pltpu.async_copy(rel_hbm.at[pl.ds(off, CO)],
                         bufb.at[pl.ds(s * CO, CO)], rb[s])

    def issue_sb(k, s):
        rvec = alow + k * CO + iota
        dst = row0 + w + MAX_REL - rvec
        valid = jnp.logical_and(rvec >= w + 1, rvec <= w + K_LEN // 2)
        idx = jnp.where(valid, dst, TRASH)
        pltpu.async_copy(bufb.at[pl.ds(s * CO, CO)], out_hbm.at[idx], sb[s])

    def wait_in(sem):
        pltpu.make_async_copy(rel_hbm.at[pl.ds(0, CO)],
                              bufa.at[pl.ds(0, CO)], sem).wait()

    def wait_out(sem):
        pltpu.make_async_copy(bufa.at[pl.ds(0, CO)],
                              out_hbm.at[pl.ds(0, CO)], sem).wait()

    for d in range(NSLOT):
        issue_ga(d, d)
        issue_rb(d, d)

    def step(k, carry):
        s = lax.rem(k, NSLOT)
        p = lax.rem(k + NSLOT - 1, NSLOT)

        for ss in range(NSLOT):
            @pl.when(jnp.logical_and(p == ss,
                                     jnp.logical_and(k >= 1, k <= NKA - 3)))
            def _():
                wait_out(wa[ss])
                issue_ga(k + 2, ss)

            @pl.when(jnp.logical_and(s == ss, k <= NKA - 1))
            def _():
                wait_in(ga[ss])
                issue_wa(k, ss)

            @pl.when(jnp.logical_and(p == ss,
                                     jnp.logical_and(k >= 1, k <= NKB - 3)))
            def _():
                wait_out(sb[ss])
                issue_rb(k + 2, ss)

            @pl.when(s == ss)
            def _():
                wait_in(rb[ss])
                issue_sb(k, ss)

        return carry

    lax.fori_loop(0, NKB, step, 0)

    for ss in range(NSLOT):
        wait_out(wa[ss])
        wait_out(sb[ss])


@functools.partial(jax.jit, static_argnames=())
def _sc_gather(rel_emb):
    mesh = plsc.VectorSubcoreMesh(core_axis_name="c", subcore_axis_name="s")
    run = pl.kernel(
        _sc_body,
        out_type=jax.ShapeDtypeStruct((Q_LEN * K_LEN + 1, D_MODEL),
                                      jnp.float32),
        mesh=mesh,
        scratch_types=(
            [pltpu.VMEM((NSLOT * CO, D_MODEL), jnp.float32),
             pltpu.VMEM((NSLOT * CO, D_MODEL), jnp.float32)]
            + [pltpu.SemaphoreType.DMA] * (4 * NSLOT)
        ),
    )
    return run(rel_emb)


def kernel(q, k, rel_emb):
    del q, k
    flat = _sc_gather(rel_emb)
    return flat[:Q_LEN * K_LEN].reshape(Q_LEN, K_LEN, D_MODEL)
